# baseline (speedup 1.0000x reference)
import functools
import math

import jax
import jax.numpy as jnp
from jax import lax
from jax.experimental import pallas as pl
from jax.experimental.pallas import tpu as pltpu
from jax.experimental.pallas import tpu_sc as plsc

DIM = 128
HALF_WORDS = DIM // 2
SCALE = math.sqrt(DIM)

NUM_CORES = 2
NUM_SUBCORES = 16
NUM_WORKERS = NUM_CORES * NUM_SUBCORES
LANES = 16

CHUNK = 128
NBUF = 4
LOOKAHEAD = 2

PACK_BLOCK = 2000


def _pack_table(table):
    V = table.shape[0]

    def body(x_ref, o_ref):
        y = x_ref[...] * SCALE
        b32 = jax.lax.bitcast_convert_type(y, jnp.int32)
        rb = b32 + jnp.int32(0x7FFF) + ((b32 >> 16) & 1)
        lo = jax.lax.shift_right_logical(rb, 16)[:, :HALF_WORDS]
        hi = rb[:, HALF_WORDS:] & jnp.int32(-65536)
        o_ref[...] = hi | lo

    return pl.pallas_call(
        body,
        grid=(V // PACK_BLOCK,),
        in_specs=[pl.BlockSpec((PACK_BLOCK, DIM), lambda i: (i, 0))],
        out_specs=pl.BlockSpec((PACK_BLOCK, HALF_WORDS), lambda i: (i, 0)),
        out_shape=jax.ShapeDtypeStruct((V, HALF_WORDS), jnp.int32),
    )(table)


def _embed_kernel(B):
    b_per_w = B // NUM_WORKERS
    n_chunks = b_per_w // CHUNK
    assert (n_chunks - 2 * LOOKAHEAD) % NBUF == 0
    mesh = plsc.VectorSubcoreMesh(
        core_axis_name="c", subcore_axis_name="s",
        num_cores=NUM_CORES, num_subcores=NUM_SUBCORES)

    @functools.partial(
        pl.kernel,
        mesh=mesh,
        compiler_params=pltpu.CompilerParams(use_tc_tiling_on_sc=False),
        out_type=jax.ShapeDtypeStruct((B, DIM), jnp.float32),
        scratch_types=[
            pltpu.VMEM((b_per_w,), jnp.int32),
            pltpu.VMEM((NBUF, CHUNK, HALF_WORDS), jnp.int32),
            pltpu.VMEM((NBUF, CHUNK, DIM), jnp.float32),
            pltpu.SemaphoreType.DMA,
            pltpu.SemaphoreType.DMA,
            pltpu.SemaphoreType.DMA,
            pltpu.SemaphoreType.DMA,
            pltpu.SemaphoreType.DMA,
            pltpu.SemaphoreType.DMA,
            pltpu.SemaphoreType.DMA,
            pltpu.SemaphoreType.DMA,
        ],
    )
    def k(x_hbm, ptab_hbm, out_hbm, idx_v, rows_v, outb_v,
          g0, g1, g2, g3, s0, s1, s2, s3):
        gsems = (g0, g1, g2, g3)
        ssems = (s0, s1, s2, s3)
        wid = lax.axis_index("s") * NUM_CORES + lax.axis_index("c")
        base = wid * b_per_w

        pltpu.sync_copy(x_hbm.at[pl.ds(base, b_per_w)], idx_v)

        def start_gather(ci, b):
            off = pl.multiple_of(ci * CHUNK, CHUNK)
            pltpu.async_copy(
                ptab_hbm.at[idx_v.at[pl.ds(off, CHUNK)]],
                rows_v.at[b], gsems[b])

        def wait_gather(b):
            pltpu.make_async_copy(
                ptab_hbm.at[pl.ds(0, CHUNK)], rows_v.at[b],
                gsems[b]).wait()

        def start_scatter(ci, b):
            off = pl.multiple_of(base + ci * CHUNK, CHUNK)
            pltpu.async_copy(outb_v.at[b], out_hbm.at[pl.ds(off, CHUNK)],
                             ssems[b])

        def wait_scatter(b):
            pltpu.make_async_copy(
                outb_v.at[b], out_hbm.at[pl.ds(0, CHUNK)], ssems[b]).wait()

        def unpack(b):
            @plsc.parallel_loop(0, CHUNK, 1)
            def _(r):
                for g in range(4):
                    w = rows_v[b, r, pl.ds(16 * g, 16)]
                    outb_v[b, r, pl.ds(16 * g, 16)] = (
                        jax.lax.bitcast_convert_type(w << 16, jnp.float32))
                    outb_v[b, r, pl.ds(HALF_WORDS + 16 * g, 16)] = (
                        jax.lax.bitcast_convert_type(
                            w & jnp.int32(-65536), jnp.float32))

        for j in range(LOOKAHEAD):
            start_gather(j, j)
        for ci in range(LOOKAHEAD):
            b = ci % NBUF
            wait_gather(b)
            unpack(b)
            start_scatter(ci, b)
            start_gather(ci + LOOKAHEAD, (ci + LOOKAHEAD) % NBUF)

        def ring_step(s, c):
            for r in range(NBUF):
                cb = LOOKAHEAD + r
                ci = s * NBUF + cb
                b = cb % NBUF
                nb = (cb + LOOKAHEAD) % NBUF
                wait_gather(b)
                unpack(b)
                start_scatter(ci, b)
                wait_scatter(nb)
                start_gather(ci + LOOKAHEAD, nb)
            return c

        lax.fori_loop(0, (n_chunks - 2 * LOOKAHEAD) // NBUF, ring_step, 0)

        for ci in range(n_chunks - LOOKAHEAD, n_chunks):
            b = ci % NBUF
            wait_gather(b)
            unpack(b)
            start_scatter(ci, b)
        for b in range(NBUF):
            wait_scatter(b)

    return k


def kernel(x, table):
    S, T = x.shape
    B = S * T
    flat = x.reshape(B).astype(jnp.int32)
    packed = _pack_table(table)
    out = _embed_kernel(B)(flat, packed)
    return out.reshape(S, T, DIM)


# speedup vs baseline: 1.1379x; 1.1379x over previous
import functools
import math

import jax
import jax.numpy as jnp
from jax import lax
from jax.experimental import pallas as pl
from jax.experimental.pallas import tpu as pltpu
from jax.experimental.pallas import tpu_sc as plsc

DIM = 128
SCALE = math.sqrt(DIM)

NUM_CORES = 2
NUM_SUBCORES = 16
NUM_WORKERS = NUM_CORES * NUM_SUBCORES
LANES = 16

CHUNK = 128
NBUF = 4
LOOKAHEAD = 2


def _embed_kernel(B):
    b_per_w = B // NUM_WORKERS
    n_chunks = b_per_w // CHUNK
    assert (n_chunks - LOOKAHEAD - (NBUF - LOOKAHEAD)) % NBUF == 0
    mesh = plsc.VectorSubcoreMesh(
        core_axis_name="c", subcore_axis_name="s",
        num_cores=NUM_CORES, num_subcores=NUM_SUBCORES)

    @functools.partial(
        pl.kernel,
        mesh=mesh,
        out_type=jax.ShapeDtypeStruct((B, DIM), jnp.float32),
        scratch_types=[
            pltpu.VMEM((b_per_w,), jnp.int32),
            pltpu.VMEM((NBUF, CHUNK, DIM), jnp.float32),
            pltpu.SemaphoreType.DMA,
            pltpu.SemaphoreType.DMA,
            pltpu.SemaphoreType.DMA,
            pltpu.SemaphoreType.DMA,
            pltpu.SemaphoreType.DMA,
            pltpu.SemaphoreType.DMA,
            pltpu.SemaphoreType.DMA,
            pltpu.SemaphoreType.DMA,
        ],
    )
    def k(x_hbm, table_hbm, out_hbm, idx_v, rows_v,
          g0, g1, g2, g3, s0, s1, s2, s3):
        gsems = (g0, g1, g2, g3)
        ssems = (s0, s1, s2, s3)
        wid = lax.axis_index("s") * NUM_CORES + lax.axis_index("c")
        base = wid * b_per_w

        pltpu.sync_copy(x_hbm.at[pl.ds(base, b_per_w)], idx_v)

        def start_gather(ci, b):
            off = pl.multiple_of(ci * CHUNK, CHUNK)
            pltpu.async_copy(
                table_hbm.at[idx_v.at[pl.ds(off, CHUNK)]],
                rows_v.at[b], gsems[b])

        def wait_gather(b):
            pltpu.make_async_copy(
                table_hbm.at[pl.ds(0, CHUNK)], rows_v.at[b],
                gsems[b]).wait()

        def start_scatter(ci, b):
            off = pl.multiple_of(base + ci * CHUNK, CHUNK)
            pltpu.async_copy(rows_v.at[b], out_hbm.at[pl.ds(off, CHUNK)],
                             ssems[b])

        def wait_scatter(b):
            pltpu.make_async_copy(
                rows_v.at[b], out_hbm.at[pl.ds(0, CHUNK)], ssems[b]).wait()

        def scale(b):
            @plsc.parallel_loop(0, CHUNK, 2)
            def _(r):
                for u in range(2):
                    for d in range(DIM // LANES):
                        sl = pl.ds(d * LANES, LANES)
                        rows_v[b, r + u, sl] = rows_v[b, r + u, sl] * SCALE

        for j in range(LOOKAHEAD):
            start_gather(j, j)
        for ci in range(LOOKAHEAD):
            b = ci % NBUF
            wait_gather(b)
            scale(b)
            start_scatter(ci, b)
            start_gather(ci + LOOKAHEAD, (ci + LOOKAHEAD) % NBUF)

        def ring_step(s, c):
            for r in range(NBUF):
                cb = LOOKAHEAD + r
                ci = s * NBUF + cb
                b = cb % NBUF
                nb = (cb + LOOKAHEAD) % NBUF
                wait_gather(b)
                scale(b)
                start_scatter(ci, b)
                wait_scatter(nb)
                start_gather(ci + LOOKAHEAD, nb)
            return c

        lax.fori_loop(0, (n_chunks - 2 * LOOKAHEAD) // NBUF, ring_step, 0)

        for ci in range(n_chunks - LOOKAHEAD, n_chunks):
            b = ci % NBUF
            wait_gather(b)
            scale(b)
            start_scatter(ci, b)
        for b in range(NBUF):
            wait_scatter(b)

    return k


def kernel(x, table):
    S, T = x.shape
    B = S * T
    flat = x.reshape(B).astype(jnp.int32)
    out = _embed_kernel(B)(flat, table)
    return out.reshape(S, T, DIM)
